# baseline (speedup 1.0000x reference)
import functools

import jax
import jax.numpy as jnp
from jax import lax
from jax.experimental import pallas as pl
from jax.experimental.pallas import tpu as pltpu
from jax.experimental.pallas import tpu_sc as plsc

M = 50000
NV = 50000
NUM_NODES = M + NV
DIM = 64
EP = 600000
EN = 200000
B = 16384
NEG = 4
REG = 1e-4

_C = 2048
_G = 128
_CB = _C + _G + 16
_NR = 25000
_NRP = _NR + 8
_RPT = _NR // 16
_MP = M + 48
_MDUMP = M


def _mesh():
    return plsc.VectorSubcoreMesh(core_axis_name="c", subcore_axis_name="s")


_SC_PARAMS = pltpu.CompilerParams(needs_layout_passes=False,
                                  use_tc_tiling_on_sc=False)


def _zero16i():
    return jnp.zeros((16,), jnp.int32)


def _make_deg():
    cfgs = ((EP, 293), (EN, 98))

    @functools.partial(
        pl.kernel,
        out_type=[jax.ShapeDtypeStruct((NUM_NODES,), jnp.float32),
                  jax.ShapeDtypeStruct((NUM_NODES,), jnp.float32)],
        mesh=_mesh(),
        compiler_params=_SC_PARAMS,
        scratch_types=[
            pltpu.VMEM((2 * _C,), jnp.int32),
            pltpu.VMEM((_CB,), jnp.int32),
            pltpu.VMEM((1, _G), jnp.int32),
            pltpu.VMEM((_G,), jnp.float32),
            pltpu.VMEM((5000,), jnp.float32),
            pltpu.VMEM_SHARED((_MP,), jnp.float32),
            pltpu.SemaphoreType.DMA,
            pltpu.SemaphoreType.DMA,
        ],
    )
    def deg_kernel(dst_p, dst_n, zeros5k, ones1, deg_p, deg_n,
                   dstb, locb, sidx, onesb, dbuf, acc, sd0, sd1):
        cid = lax.axis_index("c")
        tid = lax.axis_index("s")
        base_n = cid * M
        pltpu.sync_copy(ones1, onesb)
        ssems = (sd0, sd1)

        for g, (eph, nchunks) in enumerate(cfgs):
            dste = (dst_p, dst_n)[g]
            out = (deg_p, deg_n)[g]
            ebase = (1 - cid) * eph

            @pl.when(tid < 10)
            def _zero():
                pltpu.sync_copy(zeros5k, dbuf)
                pltpu.sync_copy(dbuf, acc.at[pl.ds(tid * 5000, 5000)])
            plsc.subcore_barrier()

            def fire_stage(k, slot):
                pltpu.async_copy(dste.at[pl.ds(ebase + k * _C, _C)],
                                 dstb.at[pl.ds(slot * _C, _C)], ssems[slot])

            def wait_stage(k, slot):
                pltpu.make_async_copy(dste.at[pl.ds(ebase + k * _C, _C)],
                                      dstb.at[pl.ds(slot * _C, _C)],
                                      ssems[slot]).wait()

            @pl.when(tid < nchunks)
            def _prime():
                fire_stage(tid, 0)

            kkmax = -(-nchunks // 16)

            def kkbody(i2, _):
              for slot in range(2):
                kk = 2 * i2 + slot
                k = kk * 16 + tid
                knext = k + 16

                @pl.when(knext < nchunks)
                def _firenext():
                    fire_stage(knext, 1 - slot)

                @pl.when(k < nchunks)
                def _process():
                    wait_stage(k, slot)

                    def fbody(j, wp):
                        vd = dstb[pl.ds(slot * _C + j * 16, 16)]
                        m = (vd >= base_n) & (vd < base_n + M)
                        mi = m.astype(jnp.int32)
                        c = plsc.cumsum(mi)
                        pos = jnp.where(m, wp + c - 1, _CB - 1)
                        plsc.store_scatter(locb, [pos], vd - base_n)
                        return wp + jnp.sum(mi)

                    wp = lax.fori_loop(0, _C // 16, fbody, 0)
                    for q in range(8):
                        locb[pl.ds(wp + q * 16, 16)] = (
                            _zero16i() + _MDUMP)
                    nb = (wp + _G - 1) // _G

                    def bbody(b, _):
                        for q in range(8):
                            sidx[0, pl.ds(q * 16, 16)] = (
                                locb[pl.ds(b * _G + q * 16, 16)])
                        pltpu.sync_copy(onesb, acc.at[sidx.at[0]], add=True)
                        return 0

                    lax.fori_loop(0, nb, bbody, 0)
              return 0

            lax.fori_loop(0, (kkmax + 1) // 2, kkbody, 0)
            plsc.subcore_barrier()

            @pl.when(tid < 10)
            def _writeout():
                pltpu.sync_copy(acc.at[pl.ds(tid * 5000, 5000)], dbuf)
                pltpu.sync_copy(dbuf,
                                out.at[pl.ds(base_n + tid * 5000, 5000)])
            plsc.subcore_barrier()

    return deg_kernel


def _make_spmm(eph, nchunks):
    kkmax = -(-nchunks // 16)
    ranges = (20000, 20000, 10000)

    @functools.partial(
        pl.kernel,
        out_type=jax.ShapeDtypeStruct((NUM_NODES, DIM), jnp.float32),
        mesh=_mesh(),
        compiler_params=_SC_PARAMS,
        scratch_types=[
            pltpu.VMEM((2 * _C,), jnp.int32),
            pltpu.VMEM((2 * _C,), jnp.int32),
            pltpu.VMEM((_CB,), jnp.int32),
            pltpu.VMEM((_CB,), jnp.int32),
            pltpu.VMEM((4, _G), jnp.int32),
            pltpu.VMEM((_G, DIM), jnp.float32),
            pltpu.VMEM((_G, DIM), jnp.float32),
            pltpu.VMEM((_G, DIM), jnp.float32),
            pltpu.VMEM((_G, DIM), jnp.float32),
            pltpu.VMEM_SHARED((20008, DIM), jnp.float32),
            pltpu.SemaphoreType.DMA,
            pltpu.SemaphoreType.DMA,
            pltpu.SemaphoreType.DMA,
            pltpu.SemaphoreType.DMA,
            pltpu.SemaphoreType.DMA,
            pltpu.SemaphoreType.DMA,
            pltpu.SemaphoreType.DMA,
            pltpu.SemaphoreType.DMA,
            pltpu.SemaphoreType.DMA,
            pltpu.SemaphoreType.DMA,
            pltpu.SemaphoreType.DMA,
            pltpu.SemaphoreType.DMA,
        ],
    )
    def spmm(xs, srce, dste, zrows, out,
             dstb, srcb, idxb, locb, sidx, rows0, rows1, rows2, rows3, acc,
             g0, g1, g2, g3, c0, c1, c2, c3, sd0, ss0, sd1, ss1):
        cid = lax.axis_index("c")
        tid = lax.axis_index("s")
        ebase = (1 - cid) * eph
        gsems = (g0, g1, g2, g3)
        csems = (c0, c1, c2, c3)
        rowsb = (rows0, rows1, rows2, rows3)
        dsems = (sd0, sd1)
        ssems = (ss0, ss1)

        def fire_stage(k, slot):
            pltpu.async_copy(dste.at[pl.ds(ebase + k * _C, _C)],
                             dstb.at[pl.ds(slot * _C, _C)], dsems[slot])
            pltpu.async_copy(srce.at[pl.ds(ebase + k * _C, _C)],
                             srcb.at[pl.ds(slot * _C, _C)], ssems[slot])

        def wait_stage(k, slot):
            pltpu.make_async_copy(dste.at[pl.ds(ebase + k * _C, _C)],
                                  dstb.at[pl.ds(slot * _C, _C)],
                                  dsems[slot]).wait()
            pltpu.make_async_copy(srce.at[pl.ds(ebase + k * _C, _C)],
                                  srcb.at[pl.ds(slot * _C, _C)],
                                  ssems[slot]).wait()

        for r, nrows in enumerate(ranges):
            lo = cid * M + r * 20000
            full = nrows // _G
            rem = nrows - full * _G

            pltpu.sync_copy(zrows, rows1)
            for jj in range(-(-(full + 1) // 16)):
                j = jj * 16 + tid

                @pl.when(j < full)
                def _z128():
                    pltpu.sync_copy(rows1, acc.at[pl.ds(j * _G, _G), :])

                @pl.when(j == full)
                def _zrem():
                    pltpu.sync_copy(rows1.at[pl.ds(0, rem), :],
                                    acc.at[pl.ds(full * _G, rem), :])
            plsc.subcore_barrier()

            @pl.when(tid < nchunks)
            def _prime():
                fire_stage(tid, 0)

            def kkbody(i2, _):
              for slot in range(2):
                kk = 2 * i2 + slot
                k = kk * 16 + tid
                knext = k + 16

                @pl.when(knext < nchunks)
                def _firenext():
                    fire_stage(knext, 1 - slot)

                @pl.when(k < nchunks)
                def _process():
                    wait_stage(k, slot)

                    def fbody(j, wp):
                        vd = dstb[pl.ds(slot * _C + j * 16, 16)]
                        vs = srcb[pl.ds(slot * _C + j * 16, 16)]
                        m = (vd >= lo) & (vd < lo + nrows)
                        mi = m.astype(jnp.int32)
                        c = plsc.cumsum(mi)
                        pos = jnp.where(m, wp + c - 1, _CB - 1)
                        plsc.store_scatter(idxb, [pos], vs)
                        plsc.store_scatter(locb, [pos], vd - lo)
                        return wp + jnp.sum(mi)

                    wp = lax.fori_loop(0, _C // 16, fbody, 0)
                    for q in range(8):
                        idxb[pl.ds(wp + q * 16, 16)] = _zero16i()
                        locb[pl.ds(wp + q * 16, 16)] = _zero16i() + 20000
                    nb = (wp + _G - 1) // _G

                    def fire_g(b, d):
                        pltpu.async_copy(
                            xs.at[idxb.at[pl.ds(b * _G, _G)]],
                            rowsb[d], gsems[d])

                    def wait_g(b, d):
                        pltpu.make_async_copy(
                            xs.at[idxb.at[pl.ds(b * _G, _G)]],
                            rowsb[d], gsems[d]).wait()

                    def fire_c(d):
                        pltpu.async_copy(rowsb[d], acc.at[sidx.at[d]],
                                         csems[d], add=True)

                    def wait_c(d):
                        pltpu.make_async_copy(rowsb[d], acc.at[sidx.at[d]],
                                              csems[d]).wait()

                    for d in range(3):
                        @pl.when(d < nb)
                        def _pg():
                            fire_g(d, d)

                    def qbody(i, _):
                        for d in range(4):
                            b = 4 * i + d

                            @pl.when(b < nb)
                            def _drain():
                                wait_g(b, d)
                                for q in range(8):
                                    sidx[d, pl.ds(q * 16, 16)] = (
                                        locb[pl.ds(b * _G + q * 16, 16)])
                                fire_c(d)
                                dprev = (d - 1) % 4

                                @pl.when((b == 0) & (3 < nb))
                                def _g3():
                                    fire_g(3, 3)

                                @pl.when((b >= 1) & (b + 3 < nb))
                                def _refill():
                                    wait_c(dprev)
                                    fire_g(b + 3, dprev)
                        return 0

                    lax.fori_loop(0, (nb + 3) // 4, qbody, 0)
                    for d in range(4):
                        @pl.when(d < nb)
                        def _dc():
                            wait_c(d)
              return 0

            lax.fori_loop(0, (kkmax + 1) // 2, kkbody, 0)
            plsc.subcore_barrier()
            for jj in range(-(-(full + 1) // 16)):
                j = jj * 16 + tid

                @pl.when(j < full)
                def _w128():
                    pltpu.sync_copy(acc.at[pl.ds(j * _G, _G), :], rows0)
                    pltpu.sync_copy(rows0, out.at[pl.ds(lo + j * _G, _G), :])

                @pl.when(j == full)
                def _wrem():
                    pltpu.sync_copy(acc.at[pl.ds(full * _G, rem), :],
                                    rows1.at[pl.ds(0, rem), :])
                    pltpu.sync_copy(rows1.at[pl.ds(0, rem), :],
                                    out.at[pl.ds(lo + full * _G, rem), :])
            plsc.subcore_barrier()

    return spmm


_spmm_p = _make_spmm(EP, 293)
_spmm_n = _make_spmm(EN, 98)


_NGATHER = B * (2 + NEG)
_BPW = _NGATHER // 32
_NBATCH = _BPW // _G


def _make_gather():
    @functools.partial(
        pl.kernel,
        out_type=jax.ShapeDtypeStruct((_NGATHER, DIM), jnp.float32),
        mesh=_mesh(),
        compiler_params=_SC_PARAMS,
        scratch_types=[
            pltpu.VMEM((_BPW,), jnp.int32),
            pltpu.VMEM((_G, DIM), jnp.float32),
            pltpu.VMEM((_G, DIM), jnp.float32),
            pltpu.SemaphoreType.DMA,
            pltpu.SemaphoreType.DMA,
            pltpu.SemaphoreType.DMA,
            pltpu.SemaphoreType.DMA,
        ],
    )
    def gather(emb, idx, out, idxv, rows0, rows1, g0, g1, w0, w1):
        cid = lax.axis_index("c")
        tid = lax.axis_index("s")
        wid = tid * 2 + cid
        base = wid * _BPW
        pltpu.sync_copy(idx.at[pl.ds(base, _BPW)], idxv)
        rowsb = (rows0, rows1)
        gsems = (g0, g1)
        wsems = (w0, w1)

        def fire_g(b, d):
            pltpu.async_copy(emb.at[idxv.at[pl.ds(b * _G, _G)]],
                             rowsb[d], gsems[d])

        def wait_g(b, d):
            pltpu.make_async_copy(emb.at[idxv.at[pl.ds(b * _G, _G)]],
                                  rowsb[d], gsems[d]).wait()

        def fire_w(b, d):
            pltpu.async_copy(rowsb[d],
                             out.at[pl.ds(base + b * _G, _G), :], wsems[d])

        def wait_w(b, d):
            pltpu.make_async_copy(rowsb[d],
                                  out.at[pl.ds(base + b * _G, _G), :],
                                  wsems[d]).wait()

        fire_g(0, 0)
        fire_g(1, 1)
        for b in range(_NBATCH):
            d = b % 2
            wait_g(b, d)
            fire_w(b, d)
            if b + 2 < _NBATCH:
                wait_w(b, d)
                fire_g(b + 2, d)
        wait_w(_NBATCH - 2, 0)
        wait_w(_NBATCH - 1, 1)

    return gather


_gather = _make_gather()


_NB = 2000


def _scale0_body(degp, degn, e1, disp, disn, s0p, s0n):
    dgp = degp[...]
    dgn = degn[...]
    dp = jnp.where(dgp > 0, lax.rsqrt(jnp.maximum(dgp, 1.0)), 0.0)
    dn = jnp.where(dgn > 0, lax.rsqrt(jnp.maximum(dgn, 1.0)), 0.0)
    disp[...] = dp
    disn[...] = dn
    e = e1[...]
    s0p[...] = dp * e
    s0n[...] = dn * e


def _scale0(deg_p, deg_n, E1):
    n_blocks = NUM_NODES // _NB
    col = pl.BlockSpec((_NB, 1), lambda i: (i, 0))
    row = pl.BlockSpec((_NB, DIM), lambda i: (i, 0))
    return pl.pallas_call(
        _scale0_body,
        grid=(n_blocks,),
        in_specs=[col, col, row],
        out_specs=[col, col, row, row],
        out_shape=[jax.ShapeDtypeStruct((NUM_NODES, 1), jnp.float32),
                   jax.ShapeDtypeStruct((NUM_NODES, 1), jnp.float32),
                   jax.ShapeDtypeStruct((NUM_NODES, DIM), jnp.float32),
                   jax.ShapeDtypeStruct((NUM_NODES, DIM), jnp.float32)],
    )(deg_p.reshape(NUM_NODES, 1), deg_n.reshape(NUM_NODES, 1), E1)


def _mid_body(t1, d, s1):
    dv = d[...]
    s1[...] = dv * dv * t1[...]


def _mid(t1, dis):
    n_blocks = NUM_NODES // _NB
    row = pl.BlockSpec((_NB, DIM), lambda i: (i, 0))
    col = pl.BlockSpec((_NB, 1), lambda i: (i, 0))
    return pl.pallas_call(
        _mid_body,
        grid=(n_blocks,),
        in_specs=[row, col],
        out_specs=row,
        out_shape=jax.ShapeDtypeStruct((NUM_NODES, DIM), jnp.float32),
    )(t1, dis)


def _fuse_emb_body(e1, e2, t1p, t2p, dp, t1n, t2n, dn, attn_w, attn_b, q_w,
                   emb_out):
    z_p = (e1[...] + dp[...] * (t1p[...] + t2p[...])) * (1.0 / 3.0)
    z_n = (e2[...] + dn[...] * (t1n[...] + t2n[...])) * (1.0 / 3.0)
    w_mat = attn_w[...]
    b_vec = attn_b[...]
    q_vec = q_w[...]
    h_p = jnp.tanh(jnp.dot(z_p, w_mat, preferred_element_type=jnp.float32)
                   + b_vec)
    h_n = jnp.tanh(jnp.dot(z_n, w_mat, preferred_element_type=jnp.float32)
                   + b_vec)
    w_p = jnp.dot(h_p, q_vec, preferred_element_type=jnp.float32)
    w_n = jnp.dot(h_n, q_vec, preferred_element_type=jnp.float32)
    mx = jnp.maximum(w_p, w_n)
    ep = jnp.exp(w_p - mx)
    en = jnp.exp(w_n - mx)
    a = ep / (ep + en)
    emb_out[...] = a * z_p + (1.0 - a) * z_n


def _fuse_emb(E1, E2, t1p, t2p, dis_p, t1n, t2n, dis_n, attn_W, attn_b, q_W):
    n_blocks = NUM_NODES // _NB
    row_spec = pl.BlockSpec((_NB, DIM), lambda i: (i, 0))
    col_spec = pl.BlockSpec((_NB, 1), lambda i: (i, 0))
    full2 = pl.BlockSpec((DIM, DIM), lambda i: (0, 0))
    fullb = pl.BlockSpec((1, DIM), lambda i: (0, 0))
    fullq = pl.BlockSpec((DIM, 1), lambda i: (0, 0))
    return pl.pallas_call(
        _fuse_emb_body,
        grid=(n_blocks,),
        in_specs=[row_spec, row_spec, row_spec, row_spec, col_spec,
                  row_spec, row_spec, col_spec, full2, fullb, fullq],
        out_specs=row_spec,
        out_shape=jax.ShapeDtypeStruct((NUM_NODES, DIM), jnp.float32),
    )(E1, E2, t1p, t2p, dis_p, t1n, t2n, dis_n, attn_W,
      attn_b.reshape(1, DIM), q_W)


_BB = 2048


def _log_sigmoid(x):
    t = -jnp.abs(x)
    return jnp.where(x < 0, x, 0.0) - jnp.log1p(jnp.exp(t))


def _loss_body(u_r, v_r, n_r, w_r, out_ref):
    i = pl.program_id(0)

    @pl.when(i == 0)
    def _init():
        out_ref[...] = jnp.zeros((1, 1), jnp.float32)

    u_ = u_r[...]
    v_ = v_r[...]
    nb = n_r[...]
    wv = w_r[...]
    pos = jnp.sum(u_ * v_, axis=1, keepdims=True)
    negsum = jnp.sum(u_[None, :, :] * nb, axis=2)
    coef = -0.5 * jnp.sign(wv) + 1.5
    x = coef * pos - negsum.T
    bpr = jnp.sum(_log_sigmoid(x))
    reg = jnp.sum(u_ * u_) + jnp.sum(v_ * v_) + jnp.sum(nb * nb)
    out_ref[...] += (-bpr + REG * reg).reshape(1, 1)


def _bpr_loss(u_, v_, n3, w):
    n_blocks = B // _BB
    row_spec = pl.BlockSpec((_BB, DIM), lambda i: (i, 0))
    n_spec = pl.BlockSpec((NEG, _BB, DIM), lambda i: (0, i, 0))
    w_spec = pl.BlockSpec((_BB, 1), lambda i: (i, 0))
    out = pl.pallas_call(
        _loss_body,
        grid=(n_blocks,),
        in_specs=[row_spec, row_spec, n_spec, w_spec],
        out_specs=pl.BlockSpec((1, 1), lambda i: (0, 0)),
        out_shape=jax.ShapeDtypeStruct((1, 1), jnp.float32),
    )(u_, v_, n3, w.reshape(B, 1))
    return out[0, 0]


def kernel(E1, E2, attn_W, attn_b, q_W, w, u, v, n, edge_index_p,
           edge_index_n):
    ipad = jnp.full((_C,), -1, jnp.int32)
    zpad = jnp.zeros((_C,), jnp.int32)
    src_p = jnp.concatenate([edge_index_p[0], zpad])
    dst_p = jnp.concatenate([edge_index_p[1], ipad])
    src_n = jnp.concatenate([edge_index_n[0], zpad])
    dst_n = jnp.concatenate([edge_index_n[1], ipad])

    zeros5k = jnp.zeros((5000,), jnp.float32)
    ones1 = jnp.ones((_G,), jnp.float32)
    zrows = jnp.zeros((_G, DIM), jnp.float32)

    deg_p, deg_n = _make_deg()(dst_p, dst_n, zeros5k, ones1)
    dis_p, dis_n, s0p, s0n = _scale0(deg_p, deg_n, E1)

    t1p = _spmm_p(s0p, src_p, dst_p, zrows)
    t2p = _spmm_p(_mid(t1p, dis_p), src_p, dst_p, zrows)
    t1n = _spmm_n(s0n, src_n, dst_n, zrows)
    t2n = _spmm_n(_mid(t1n, dis_n), src_n, dst_n, zrows)

    emb = _fuse_emb(E1, E2, t1p, t2p, dis_p, t1n, t2n, dis_n,
                    attn_W, attn_b, q_W)

    idxall = jnp.concatenate([u, v, n.T.reshape(-1)]).astype(jnp.int32)
    g = _gather(emb, idxall)
    u_ = g[0:B]
    v_ = g[B:2 * B]
    n3 = g[2 * B:].reshape(NEG, B, DIM)
    return _bpr_loss(u_, v_, n3, w)


# speedup vs baseline: 1.6201x; 1.6201x over previous
import functools

import jax
import jax.numpy as jnp
from jax import lax
from jax.experimental import pallas as pl
from jax.experimental.pallas import tpu as pltpu
from jax.experimental.pallas import tpu_sc as plsc

M = 50000
NV = 50000
NUM_NODES = M + NV
DIM = 64
EP = 600000
EN = 200000
B = 16384
NEG = 4
REG = 1e-4

_C = 2048
_G = 128
_CB = _C + _G + 16
_NR = 25000
_NRP = _NR + 8
_RPT = _NR // 16
_MP = M + 48
_MDUMP = M


def _mesh():
    return plsc.VectorSubcoreMesh(core_axis_name="c", subcore_axis_name="s")


_SC_PARAMS = pltpu.CompilerParams(needs_layout_passes=False,
                                  use_tc_tiling_on_sc=False)


def _zero16i():
    return jnp.zeros((16,), jnp.int32)


def _make_deg():
    cfgs = ((EP, 293), (EN, 98))

    @functools.partial(
        pl.kernel,
        out_type=[jax.ShapeDtypeStruct((NUM_NODES,), jnp.float32),
                  jax.ShapeDtypeStruct((NUM_NODES,), jnp.float32)],
        mesh=_mesh(),
        compiler_params=_SC_PARAMS,
        scratch_types=[
            pltpu.VMEM((2 * _C,), jnp.int32),
            pltpu.VMEM((_CB,), jnp.int32),
            pltpu.VMEM((1, _G), jnp.int32),
            pltpu.VMEM((_G,), jnp.float32),
            pltpu.VMEM((5000,), jnp.float32),
            pltpu.VMEM_SHARED((_MP,), jnp.float32),
            pltpu.SemaphoreType.DMA,
            pltpu.SemaphoreType.DMA,
        ],
    )
    def deg_kernel(dst_p, dst_n, zeros5k, ones1, deg_p, deg_n,
                   dstb, locb, sidx, onesb, dbuf, acc, sd0, sd1):
        cid = lax.axis_index("c")
        tid = lax.axis_index("s")
        base_n = cid * M
        pltpu.sync_copy(ones1, onesb)
        ssems = (sd0, sd1)

        for g, (eph, nchunks) in enumerate(cfgs):
            dste = (dst_p, dst_n)[g]
            out = (deg_p, deg_n)[g]
            ebase = (1 - cid) * eph

            @pl.when(tid < 10)
            def _zero():
                pltpu.sync_copy(zeros5k, dbuf)
                pltpu.sync_copy(dbuf, acc.at[pl.ds(tid * 5000, 5000)])
            plsc.subcore_barrier()

            def fire_stage(k, slot):
                pltpu.async_copy(dste.at[pl.ds(ebase + k * _C, _C)],
                                 dstb.at[pl.ds(slot * _C, _C)], ssems[slot])

            def wait_stage(k, slot):
                pltpu.make_async_copy(dste.at[pl.ds(ebase + k * _C, _C)],
                                      dstb.at[pl.ds(slot * _C, _C)],
                                      ssems[slot]).wait()

            @pl.when(tid < nchunks)
            def _prime():
                fire_stage(tid, 0)

            kkmax = -(-nchunks // 16)

            def kkbody(i2, _):
              for slot in range(2):
                kk = 2 * i2 + slot
                k = kk * 16 + tid
                knext = k + 16

                @pl.when(knext < nchunks)
                def _firenext():
                    fire_stage(knext, 1 - slot)

                @pl.when(k < nchunks)
                def _process():
                    wait_stage(k, slot)

                    def fbody(j, wp):
                        vd = dstb[pl.ds(slot * _C + j * 16, 16)]
                        m = (vd >= base_n) & (vd < base_n + M)
                        mi = m.astype(jnp.int32)
                        c = plsc.cumsum(mi)
                        pos = jnp.where(m, wp + c - 1, _CB - 1)
                        plsc.store_scatter(locb, [pos], vd - base_n)
                        return wp + jnp.sum(mi)

                    wp = lax.fori_loop(0, _C // 16, fbody, 0)
                    for q in range(8):
                        locb[pl.ds(wp + q * 16, 16)] = (
                            _zero16i() + _MDUMP)
                    nb = (wp + _G - 1) // _G

                    def bbody(b, _):
                        for q in range(8):
                            sidx[0, pl.ds(q * 16, 16)] = (
                                locb[pl.ds(b * _G + q * 16, 16)])
                        pltpu.sync_copy(onesb, acc.at[sidx.at[0]], add=True)
                        return 0

                    lax.fori_loop(0, nb, bbody, 0)
              return 0

            lax.fori_loop(0, (kkmax + 1) // 2, kkbody, 0)
            plsc.subcore_barrier()

            @pl.when(tid < 10)
            def _writeout():
                pltpu.sync_copy(acc.at[pl.ds(tid * 5000, 5000)], dbuf)
                pltpu.sync_copy(dbuf,
                                out.at[pl.ds(base_n + tid * 5000, 5000)])
            plsc.subcore_barrier()

    return deg_kernel


def _make_spmm(eph, nchunks):
    kkmax = -(-nchunks // 16)

    @functools.partial(
        pl.kernel,
        out_type=jax.ShapeDtypeStruct((NUM_NODES, DIM), jnp.float32),
        mesh=_mesh(),
        compiler_params=_SC_PARAMS,
        scratch_types=[
            pltpu.VMEM((2 * _C,), jnp.int32),
            pltpu.VMEM((2 * _C,), jnp.int32),
            pltpu.VMEM((_CB,), jnp.int32),
            pltpu.VMEM((_CB,), jnp.int32),
            pltpu.VMEM((2, _G), jnp.int32),
            pltpu.VMEM((_G, DIM), jnp.float32),
            pltpu.VMEM((_G, DIM), jnp.float32),
            pltpu.VMEM_SHARED((_NRP, DIM), jnp.float32),
            pltpu.SemaphoreType.DMA,
            pltpu.SemaphoreType.DMA,
            pltpu.SemaphoreType.DMA,
            pltpu.SemaphoreType.DMA,
            pltpu.SemaphoreType.DMA,
            pltpu.SemaphoreType.DMA,
            pltpu.SemaphoreType.DMA,
            pltpu.SemaphoreType.DMA,
        ],
    )
    def spmm(xs, srce, dste, zrows, out,
             dstb, srcb, idxb, locb, sidx, rows0, rows1, acc,
             g0, g1, c0, c1, sd0, ss0, sd1, ss1):
        cid = lax.axis_index("c")
        tid = lax.axis_index("s")
        ebase = (1 - cid) * eph
        gsems = (g0, g1)
        csems = (c0, c1)
        rowsb = (rows0, rows1)
        dsems = (sd0, sd1)
        ssems = (ss0, ss1)

        def fire_stage(k, slot):
            pltpu.async_copy(dste.at[pl.ds(ebase + k * _C, _C)],
                             dstb.at[pl.ds(slot * _C, _C)], dsems[slot])
            pltpu.async_copy(srce.at[pl.ds(ebase + k * _C, _C)],
                             srcb.at[pl.ds(slot * _C, _C)], ssems[slot])

        def wait_stage(k, slot):
            pltpu.make_async_copy(dste.at[pl.ds(ebase + k * _C, _C)],
                                  dstb.at[pl.ds(slot * _C, _C)],
                                  dsems[slot]).wait()
            pltpu.make_async_copy(srce.at[pl.ds(ebase + k * _C, _C)],
                                  srcb.at[pl.ds(slot * _C, _C)],
                                  ssems[slot]).wait()

        for r in range(2):
            lo = cid * M + r * _NR

            pltpu.sync_copy(zrows, rows1)
            for jj in range(13):
                j = jj * 16 + tid

                @pl.when(j < 195)
                def _z128():
                    pltpu.sync_copy(rows1, acc.at[pl.ds(j * _G, _G), :])

                @pl.when(j == 195)
                def _z40():
                    pltpu.sync_copy(rows1.at[pl.ds(0, 40), :],
                                    acc.at[pl.ds(195 * _G, 40), :])
            plsc.subcore_barrier()

            @pl.when(tid < nchunks)
            def _prime():
                fire_stage(tid, 0)

            def kkbody(i2, _):
              for slot in range(2):
                kk = 2 * i2 + slot
                k = kk * 16 + tid
                knext = k + 16

                @pl.when(knext < nchunks)
                def _firenext():
                    fire_stage(knext, 1 - slot)

                @pl.when(k < nchunks)
                def _process():
                    wait_stage(k, slot)

                    def fbody(j, wp):
                        vd = dstb[pl.ds(slot * _C + j * 16, 16)]
                        vs = srcb[pl.ds(slot * _C + j * 16, 16)]
                        m = (vd >= lo) & (vd < lo + _NR)
                        mi = m.astype(jnp.int32)
                        c = plsc.cumsum(mi)
                        pos = jnp.where(m, wp + c - 1, _CB - 1)
                        plsc.store_scatter(idxb, [pos], vs)
                        plsc.store_scatter(locb, [pos], vd - lo)
                        return wp + jnp.sum(mi)

                    wp = lax.fori_loop(0, _C // 16, fbody, 0)
                    for q in range(8):
                        idxb[pl.ds(wp + q * 16, 16)] = _zero16i()
                        locb[pl.ds(wp + q * 16, 16)] = _zero16i() + _NR
                    nb = (wp + _G - 1) // _G

                    def fire_g(b, d):
                        pltpu.async_copy(
                            xs.at[idxb.at[pl.ds(b * _G, _G)]],
                            rowsb[d], gsems[d])

                    def wait_g(b, d):
                        pltpu.make_async_copy(
                            xs.at[idxb.at[pl.ds(b * _G, _G)]],
                            rowsb[d], gsems[d]).wait()

                    def fire_c(d):
                        pltpu.async_copy(rowsb[d], acc.at[sidx.at[d]],
                                         csems[d], add=True)

                    def wait_c(d):
                        pltpu.make_async_copy(rowsb[d], acc.at[sidx.at[d]],
                                              csems[d]).wait()

                    def drain(b, d):
                        wait_g(b, d)
                        for q in range(8):
                            sidx[d, pl.ds(q * 16, 16)] = (
                                locb[pl.ds(b * _G + q * 16, 16)])
                        fire_c(d)

                    @pl.when(nb > 0)
                    def _prime_g():
                        fire_g(0, 0)

                    def pbody(i, _):
                        b0 = 2 * i

                        @pl.when(b0 + 1 < nb)
                        def _f1():
                            @pl.when(b0 >= 1)
                            def _wc1():
                                wait_c(1)
                            fire_g(b0 + 1, 1)
                        drain(b0, 0)

                        @pl.when(b0 + 2 < nb)
                        def _f2():
                            wait_c(0)
                            fire_g(b0 + 2, 0)

                        @pl.when(b0 + 1 < nb)
                        def _d1():
                            drain(b0 + 1, 1)
                        return 0

                    lax.fori_loop(0, (nb + 1) // 2, pbody, 0)
                    for d in range(2):
                        @pl.when(d < nb)
                        def _dc():
                            wait_c(d)
              return 0

            lax.fori_loop(0, (kkmax + 1) // 2, kkbody, 0)
            plsc.subcore_barrier()
            for jj in range(13):
                j = jj * 16 + tid

                @pl.when(j < 195)
                def _w128():
                    pltpu.sync_copy(acc.at[pl.ds(j * _G, _G), :], rows0)
                    pltpu.sync_copy(rows0, out.at[pl.ds(lo + j * _G, _G), :])

                @pl.when(j == 195)
                def _w40():
                    pltpu.sync_copy(acc.at[pl.ds(195 * _G, 40), :],
                                    rows1.at[pl.ds(0, 40), :])
                    pltpu.sync_copy(rows1.at[pl.ds(0, 40), :],
                                    out.at[pl.ds(lo + 195 * _G, 40), :])
            plsc.subcore_barrier()

    return spmm


_spmm_p = _make_spmm(EP, 293)
_spmm_n = _make_spmm(EN, 98)


_NGATHER = B * (2 + NEG)
_BPW = _NGATHER // 32
_NBATCH = _BPW // _G


def _make_gather():
    @functools.partial(
        pl.kernel,
        out_type=jax.ShapeDtypeStruct((_NGATHER, DIM), jnp.float32),
        mesh=_mesh(),
        compiler_params=_SC_PARAMS,
        scratch_types=[
            pltpu.VMEM((_BPW,), jnp.int32),
            pltpu.VMEM((_G, DIM), jnp.float32),
            pltpu.VMEM((_G, DIM), jnp.float32),
            pltpu.SemaphoreType.DMA,
            pltpu.SemaphoreType.DMA,
            pltpu.SemaphoreType.DMA,
            pltpu.SemaphoreType.DMA,
        ],
    )
    def gather(emb, idx, out, idxv, rows0, rows1, g0, g1, w0, w1):
        cid = lax.axis_index("c")
        tid = lax.axis_index("s")
        wid = tid * 2 + cid
        base = wid * _BPW
        pltpu.sync_copy(idx.at[pl.ds(base, _BPW)], idxv)
        rowsb = (rows0, rows1)
        gsems = (g0, g1)
        wsems = (w0, w1)

        def fire_g(b, d):
            pltpu.async_copy(emb.at[idxv.at[pl.ds(b * _G, _G)]],
                             rowsb[d], gsems[d])

        def wait_g(b, d):
            pltpu.make_async_copy(emb.at[idxv.at[pl.ds(b * _G, _G)]],
                                  rowsb[d], gsems[d]).wait()

        def fire_w(b, d):
            pltpu.async_copy(rowsb[d],
                             out.at[pl.ds(base + b * _G, _G), :], wsems[d])

        def wait_w(b, d):
            pltpu.make_async_copy(rowsb[d],
                                  out.at[pl.ds(base + b * _G, _G), :],
                                  wsems[d]).wait()

        fire_g(0, 0)
        fire_g(1, 1)
        for b in range(_NBATCH):
            d = b % 2
            wait_g(b, d)
            fire_w(b, d)
            if b + 2 < _NBATCH:
                wait_w(b, d)
                fire_g(b + 2, d)
        wait_w(_NBATCH - 2, 0)
        wait_w(_NBATCH - 1, 1)

    return gather


_gather = _make_gather()


_NB = 2000


def _scale0_body(degp, degn, e1, disp, disn, s0p, s0n):
    dgp = degp[...]
    dgn = degn[...]
    dp = jnp.where(dgp > 0, lax.rsqrt(jnp.maximum(dgp, 1.0)), 0.0)
    dn = jnp.where(dgn > 0, lax.rsqrt(jnp.maximum(dgn, 1.0)), 0.0)
    disp[...] = dp
    disn[...] = dn
    e = e1[...]
    s0p[...] = dp * e
    s0n[...] = dn * e


def _scale0(deg_p, deg_n, E1):
    n_blocks = NUM_NODES // _NB
    col = pl.BlockSpec((_NB, 1), lambda i: (i, 0))
    row = pl.BlockSpec((_NB, DIM), lambda i: (i, 0))
    return pl.pallas_call(
        _scale0_body,
        grid=(n_blocks,),
        in_specs=[col, col, row],
        out_specs=[col, col, row, row],
        out_shape=[jax.ShapeDtypeStruct((NUM_NODES, 1), jnp.float32),
                   jax.ShapeDtypeStruct((NUM_NODES, 1), jnp.float32),
                   jax.ShapeDtypeStruct((NUM_NODES, DIM), jnp.float32),
                   jax.ShapeDtypeStruct((NUM_NODES, DIM), jnp.float32)],
    )(deg_p.reshape(NUM_NODES, 1), deg_n.reshape(NUM_NODES, 1), E1)


def _mid_body(t1, d, s1):
    dv = d[...]
    s1[...] = dv * dv * t1[...]


def _mid(t1, dis):
    n_blocks = NUM_NODES // _NB
    row = pl.BlockSpec((_NB, DIM), lambda i: (i, 0))
    col = pl.BlockSpec((_NB, 1), lambda i: (i, 0))
    return pl.pallas_call(
        _mid_body,
        grid=(n_blocks,),
        in_specs=[row, col],
        out_specs=row,
        out_shape=jax.ShapeDtypeStruct((NUM_NODES, DIM), jnp.float32),
    )(t1, dis)


def _fuse_emb_body(e1, e2, t1p, t2p, dp, t1n, t2n, dn, attn_w, attn_b, q_w,
                   emb_out):
    z_p = (e1[...] + dp[...] * (t1p[...] + t2p[...])) * (1.0 / 3.0)
    z_n = (e2[...] + dn[...] * (t1n[...] + t2n[...])) * (1.0 / 3.0)
    w_mat = attn_w[...]
    b_vec = attn_b[...]
    q_vec = q_w[...]
    h_p = jnp.tanh(jnp.dot(z_p, w_mat, preferred_element_type=jnp.float32)
                   + b_vec)
    h_n = jnp.tanh(jnp.dot(z_n, w_mat, preferred_element_type=jnp.float32)
                   + b_vec)
    w_p = jnp.dot(h_p, q_vec, preferred_element_type=jnp.float32)
    w_n = jnp.dot(h_n, q_vec, preferred_element_type=jnp.float32)
    mx = jnp.maximum(w_p, w_n)
    ep = jnp.exp(w_p - mx)
    en = jnp.exp(w_n - mx)
    a = ep / (ep + en)
    emb_out[...] = a * z_p + (1.0 - a) * z_n


def _fuse_emb(E1, E2, t1p, t2p, dis_p, t1n, t2n, dis_n, attn_W, attn_b, q_W):
    n_blocks = NUM_NODES // _NB
    row_spec = pl.BlockSpec((_NB, DIM), lambda i: (i, 0))
    col_spec = pl.BlockSpec((_NB, 1), lambda i: (i, 0))
    full2 = pl.BlockSpec((DIM, DIM), lambda i: (0, 0))
    fullb = pl.BlockSpec((1, DIM), lambda i: (0, 0))
    fullq = pl.BlockSpec((DIM, 1), lambda i: (0, 0))
    return pl.pallas_call(
        _fuse_emb_body,
        grid=(n_blocks,),
        in_specs=[row_spec, row_spec, row_spec, row_spec, col_spec,
                  row_spec, row_spec, col_spec, full2, fullb, fullq],
        out_specs=row_spec,
        out_shape=jax.ShapeDtypeStruct((NUM_NODES, DIM), jnp.float32),
    )(E1, E2, t1p, t2p, dis_p, t1n, t2n, dis_n, attn_W,
      attn_b.reshape(1, DIM), q_W)


_BB = 2048


def _log_sigmoid(x):
    t = -jnp.abs(x)
    return jnp.where(x < 0, x, 0.0) - jnp.log1p(jnp.exp(t))


def _loss_body(u_r, v_r, n_r, w_r, out_ref):
    i = pl.program_id(0)

    @pl.when(i == 0)
    def _init():
        out_ref[...] = jnp.zeros((1, 1), jnp.float32)

    u_ = u_r[...]
    v_ = v_r[...]
    nb = n_r[...]
    wv = w_r[...]
    pos = jnp.sum(u_ * v_, axis=1, keepdims=True)
    negsum = jnp.sum(u_[None, :, :] * nb, axis=2)
    coef = -0.5 * jnp.sign(wv) + 1.5
    x = coef * pos - negsum.T
    bpr = jnp.sum(_log_sigmoid(x))
    reg = jnp.sum(u_ * u_) + jnp.sum(v_ * v_) + jnp.sum(nb * nb)
    out_ref[...] += (-bpr + REG * reg).reshape(1, 1)


def _bpr_loss(u_, v_, n3, w):
    n_blocks = B // _BB
    row_spec = pl.BlockSpec((_BB, DIM), lambda i: (i, 0))
    n_spec = pl.BlockSpec((NEG, _BB, DIM), lambda i: (0, i, 0))
    w_spec = pl.BlockSpec((_BB, 1), lambda i: (i, 0))
    out = pl.pallas_call(
        _loss_body,
        grid=(n_blocks,),
        in_specs=[row_spec, row_spec, n_spec, w_spec],
        out_specs=pl.BlockSpec((1, 1), lambda i: (0, 0)),
        out_shape=jax.ShapeDtypeStruct((1, 1), jnp.float32),
    )(u_, v_, n3, w.reshape(B, 1))
    return out[0, 0]


def kernel(E1, E2, attn_W, attn_b, q_W, w, u, v, n, edge_index_p,
           edge_index_n):
    ipad = jnp.full((_C,), -1, jnp.int32)
    zpad = jnp.zeros((_C,), jnp.int32)
    src_p = jnp.concatenate([edge_index_p[0], zpad])
    dst_p = jnp.concatenate([edge_index_p[1], ipad])
    src_n = jnp.concatenate([edge_index_n[0], zpad])
    dst_n = jnp.concatenate([edge_index_n[1], ipad])

    zeros5k = jnp.zeros((5000,), jnp.float32)
    ones1 = jnp.ones((_G,), jnp.float32)
    zrows = jnp.zeros((_G, DIM), jnp.float32)

    deg_p, deg_n = _make_deg()(dst_p, dst_n, zeros5k, ones1)
    dis_p, dis_n, s0p, s0n = _scale0(deg_p, deg_n, E1)

    t1p = _spmm_p(s0p, src_p, dst_p, zrows)
    t2p = _spmm_p(_mid(t1p, dis_p), src_p, dst_p, zrows)
    t1n = _spmm_n(s0n, src_n, dst_n, zrows)
    t2n = _spmm_n(_mid(t1n, dis_n), src_n, dst_n, zrows)

    emb = _fuse_emb(E1, E2, t1p, t2p, dis_p, t1n, t2n, dis_n,
                    attn_W, attn_b, q_W)

    idxall = jnp.concatenate([u, v, n.T.reshape(-1)]).astype(jnp.int32)
    g = _gather(emb, idxall)
    u_ = g[0:B]
    v_ = g[B:2 * B]
    n3 = g[2 * B:].reshape(NEG, B, DIM)
    return _bpr_loss(u_, v_, n3, w)


# speedup vs baseline: 2.5435x; 1.5700x over previous
import functools

import jax
import jax.numpy as jnp
from jax import lax
from jax.experimental import pallas as pl
from jax.experimental.pallas import tpu as pltpu
from jax.experimental.pallas import tpu_sc as plsc

M = 50000
NV = 50000
NUM_NODES = M + NV
DIM = 64
EP = 600000
EN = 200000
B = 16384
NEG = 4
REG = 1e-4

_C = 2048
_G = 128
_CB = _C + _G + 16
_NR = 25000
_NRP = _NR + 8
_RPT = _NR // 16
_MP = M + 48
_MDUMP = M


def _mesh():
    return plsc.VectorSubcoreMesh(core_axis_name="c", subcore_axis_name="s")


_SC_PARAMS = pltpu.CompilerParams(needs_layout_passes=False,
                                  use_tc_tiling_on_sc=False)


def _zero16i():
    return jnp.zeros((16,), jnp.int32)


def _make_deg():
    cfgs = ((EP, 293), (EN, 98))

    @functools.partial(
        pl.kernel,
        out_type=[jax.ShapeDtypeStruct((NUM_NODES,), jnp.float32),
                  jax.ShapeDtypeStruct((NUM_NODES,), jnp.float32)],
        mesh=_mesh(),
        compiler_params=_SC_PARAMS,
        scratch_types=[
            pltpu.VMEM((2 * _C,), jnp.int32),
            pltpu.VMEM((_CB,), jnp.int32),
            pltpu.VMEM((1, _G), jnp.int32),
            pltpu.VMEM((_G,), jnp.float32),
            pltpu.VMEM((5000,), jnp.float32),
            pltpu.VMEM_SHARED((_MP,), jnp.float32),
            pltpu.SemaphoreType.DMA,
            pltpu.SemaphoreType.DMA,
        ],
    )
    def deg_kernel(dst_p, dst_n, zeros5k, ones1, deg_p, deg_n,
                   dstb, locb, sidx, onesb, dbuf, acc, sd0, sd1):
        cid = lax.axis_index("c")
        tid = lax.axis_index("s")
        base_n = cid * M
        pltpu.sync_copy(ones1, onesb)
        ssems = (sd0, sd1)

        for g, (eph, nchunks) in enumerate(cfgs):
            dste = (dst_p, dst_n)[g]
            out = (deg_p, deg_n)[g]
            ebase = (1 - cid) * eph

            @pl.when(tid < 10)
            def _zero():
                pltpu.sync_copy(zeros5k, dbuf)
                pltpu.sync_copy(dbuf, acc.at[pl.ds(tid * 5000, 5000)])
            plsc.subcore_barrier()

            def fire_stage(k, slot):
                pltpu.async_copy(dste.at[pl.ds(ebase + k * _C, _C)],
                                 dstb.at[pl.ds(slot * _C, _C)], ssems[slot])

            def wait_stage(k, slot):
                pltpu.make_async_copy(dste.at[pl.ds(ebase + k * _C, _C)],
                                      dstb.at[pl.ds(slot * _C, _C)],
                                      ssems[slot]).wait()

            @pl.when(tid < nchunks)
            def _prime():
                fire_stage(tid, 0)

            kkmax = -(-nchunks // 16)

            def kkbody(i2, _):
              for slot in range(2):
                kk = 2 * i2 + slot
                k = kk * 16 + tid
                knext = k + 16

                @pl.when(knext < nchunks)
                def _firenext():
                    fire_stage(knext, 1 - slot)

                @pl.when(k < nchunks)
                def _process():
                    wait_stage(k, slot)

                    def fbody(j, wp):
                        vd = dstb[pl.ds(slot * _C + j * 16, 16)]
                        m = (vd >= base_n) & (vd < base_n + M)
                        mi = m.astype(jnp.int32)
                        c = plsc.cumsum(mi)
                        pos = jnp.where(m, wp + c - 1, _CB - 1)
                        plsc.store_scatter(locb, [pos], vd - base_n)
                        return wp + jnp.sum(mi)

                    wp = lax.fori_loop(0, _C // 16, fbody, 0)
                    for q in range(8):
                        locb[pl.ds(wp + q * 16, 16)] = (
                            _zero16i() + _MDUMP)
                    nb = (wp + _G - 1) // _G

                    def bbody(b, _):
                        for q in range(8):
                            sidx[0, pl.ds(q * 16, 16)] = (
                                locb[pl.ds(b * _G + q * 16, 16)])
                        pltpu.sync_copy(onesb, acc.at[sidx.at[0]], add=True)
                        return 0

                    lax.fori_loop(0, nb, bbody, 0)
              return 0

            lax.fori_loop(0, (kkmax + 1) // 2, kkbody, 0)
            plsc.subcore_barrier()

            @pl.when(tid < 10)
            def _writeout():
                pltpu.sync_copy(acc.at[pl.ds(tid * 5000, 5000)], dbuf)
                pltpu.sync_copy(dbuf,
                                out.at[pl.ds(base_n + tid * 5000, 5000)])
            plsc.subcore_barrier()

    return deg_kernel


def _make_spmm(eph, nchunks):
    kkmax = -(-nchunks // 16)

    @functools.partial(
        pl.kernel,
        out_type=jax.ShapeDtypeStruct((NUM_NODES, DIM), jnp.bfloat16),
        mesh=_mesh(),
        compiler_params=_SC_PARAMS,
        scratch_types=[
            pltpu.VMEM((2 * _C,), jnp.int32),
            pltpu.VMEM((2 * _C,), jnp.int32),
            pltpu.VMEM((_CB,), jnp.int32),
            pltpu.VMEM((_CB,), jnp.int32),
            pltpu.VMEM((2, _G), jnp.int32),
            pltpu.VMEM((_G, DIM), jnp.bfloat16),
            pltpu.VMEM((_G, DIM), jnp.bfloat16),
            pltpu.VMEM_SHARED((_NRP, DIM), jnp.bfloat16),
            pltpu.SemaphoreType.DMA,
            pltpu.SemaphoreType.DMA,
            pltpu.SemaphoreType.DMA,
            pltpu.SemaphoreType.DMA,
            pltpu.SemaphoreType.DMA,
            pltpu.SemaphoreType.DMA,
            pltpu.SemaphoreType.DMA,
            pltpu.SemaphoreType.DMA,
        ],
    )
    def spmm(xs, srce, dste, zrows, out,
             dstb, srcb, idxb, locb, sidx, rows0, rows1, acc,
             g0, g1, c0, c1, sd0, ss0, sd1, ss1):
        cid = lax.axis_index("c")
        tid = lax.axis_index("s")
        ebase = (1 - cid) * eph
        gsems = (g0, g1)
        csems = (c0, c1)
        rowsb = (rows0, rows1)
        dsems = (sd0, sd1)
        ssems = (ss0, ss1)

        def fire_stage(k, slot):
            pltpu.async_copy(dste.at[pl.ds(ebase + k * _C, _C)],
                             dstb.at[pl.ds(slot * _C, _C)], dsems[slot])
            pltpu.async_copy(srce.at[pl.ds(ebase + k * _C, _C)],
                             srcb.at[pl.ds(slot * _C, _C)], ssems[slot])

        def wait_stage(k, slot):
            pltpu.make_async_copy(dste.at[pl.ds(ebase + k * _C, _C)],
                                  dstb.at[pl.ds(slot * _C, _C)],
                                  dsems[slot]).wait()
            pltpu.make_async_copy(srce.at[pl.ds(ebase + k * _C, _C)],
                                  srcb.at[pl.ds(slot * _C, _C)],
                                  ssems[slot]).wait()

        for r in range(2):
            lo = cid * M + r * _NR

            pltpu.sync_copy(zrows, rows1)
            for jj in range(13):
                j = jj * 16 + tid

                @pl.when(j < 195)
                def _z128():
                    pltpu.sync_copy(rows1, acc.at[pl.ds(j * _G, _G), :])

                @pl.when(j == 195)
                def _z40():
                    pltpu.sync_copy(rows1.at[pl.ds(0, 40), :],
                                    acc.at[pl.ds(195 * _G, 40), :])
            plsc.subcore_barrier()

            @pl.when(tid < nchunks)
            def _prime():
                fire_stage(tid, 0)

            def kkbody(i2, _):
              for slot in range(2):
                kk = 2 * i2 + slot
                k = kk * 16 + tid
                knext = k + 16

                @pl.when(knext < nchunks)
                def _firenext():
                    fire_stage(knext, 1 - slot)

                @pl.when(k < nchunks)
                def _process():
                    wait_stage(k, slot)

                    def fbody(j, wp):
                        vd = dstb[pl.ds(slot * _C + j * 16, 16)]
                        vs = srcb[pl.ds(slot * _C + j * 16, 16)]
                        m = (vd >= lo) & (vd < lo + _NR)
                        mi = m.astype(jnp.int32)
                        c = plsc.cumsum(mi)
                        pos = jnp.where(m, wp + c - 1, _CB - 1)
                        plsc.store_scatter(idxb, [pos], vs)
                        plsc.store_scatter(locb, [pos], vd - lo)
                        return wp + jnp.sum(mi)

                    wp = lax.fori_loop(0, _C // 16, fbody, 0)
                    for q in range(8):
                        idxb[pl.ds(wp + q * 16, 16)] = _zero16i()
                        locb[pl.ds(wp + q * 16, 16)] = _zero16i() + _NR
                    nb = (wp + _G - 1) // _G

                    def fire_g(b, d):
                        pltpu.async_copy(
                            xs.at[idxb.at[pl.ds(b * _G, _G)]],
                            rowsb[d], gsems[d])

                    def wait_g(b, d):
                        pltpu.make_async_copy(
                            xs.at[idxb.at[pl.ds(b * _G, _G)]],
                            rowsb[d], gsems[d]).wait()

                    def fire_c(d):
                        pltpu.async_copy(rowsb[d], acc.at[sidx.at[d]],
                                         csems[d], add=True)

                    def wait_c(d):
                        pltpu.make_async_copy(rowsb[d], acc.at[sidx.at[d]],
                                              csems[d]).wait()

                    def drain(b, d):
                        wait_g(b, d)
                        for q in range(8):
                            sidx[d, pl.ds(q * 16, 16)] = (
                                locb[pl.ds(b * _G + q * 16, 16)])
                        fire_c(d)

                    @pl.when(nb > 0)
                    def _prime_g():
                        fire_g(0, 0)

                    def pbody(i, _):
                        b0 = 2 * i

                        @pl.when(b0 + 1 < nb)
                        def _f1():
                            @pl.when(b0 >= 1)
                            def _wc1():
                                wait_c(1)
                            fire_g(b0 + 1, 1)
                        drain(b0, 0)

                        @pl.when(b0 + 2 < nb)
                        def _f2():
                            wait_c(0)
                            fire_g(b0 + 2, 0)

                        @pl.when(b0 + 1 < nb)
                        def _d1():
                            drain(b0 + 1, 1)
                        return 0

                    lax.fori_loop(0, (nb + 1) // 2, pbody, 0)
                    for d in range(2):
                        @pl.when(d < nb)
                        def _dc():
                            wait_c(d)
              return 0

            lax.fori_loop(0, (kkmax + 1) // 2, kkbody, 0)
            plsc.subcore_barrier()
            for jj in range(13):
                j = jj * 16 + tid

                @pl.when(j < 195)
                def _w128():
                    pltpu.sync_copy(acc.at[pl.ds(j * _G, _G), :], rows0)
                    pltpu.sync_copy(rows0, out.at[pl.ds(lo + j * _G, _G), :])

                @pl.when(j == 195)
                def _w40():
                    pltpu.sync_copy(acc.at[pl.ds(195 * _G, 40), :],
                                    rows1.at[pl.ds(0, 40), :])
                    pltpu.sync_copy(rows1.at[pl.ds(0, 40), :],
                                    out.at[pl.ds(lo + 195 * _G, 40), :])
            plsc.subcore_barrier()

    return spmm


_spmm_p = _make_spmm(EP, 293)
_spmm_n = _make_spmm(EN, 98)


_NGATHER = B * (2 + NEG)
_BPW = _NGATHER // 32
_NBATCH = _BPW // _G


def _make_gather():
    @functools.partial(
        pl.kernel,
        out_type=jax.ShapeDtypeStruct((_NGATHER, DIM), jnp.float32),
        mesh=_mesh(),
        compiler_params=_SC_PARAMS,
        scratch_types=[
            pltpu.VMEM((_BPW,), jnp.int32),
            pltpu.VMEM((_G, DIM), jnp.float32),
            pltpu.VMEM((_G, DIM), jnp.float32),
            pltpu.SemaphoreType.DMA,
            pltpu.SemaphoreType.DMA,
            pltpu.SemaphoreType.DMA,
            pltpu.SemaphoreType.DMA,
        ],
    )
    def gather(emb, idx, out, idxv, rows0, rows1, g0, g1, w0, w1):
        cid = lax.axis_index("c")
        tid = lax.axis_index("s")
        wid = tid * 2 + cid
        base = wid * _BPW
        pltpu.sync_copy(idx.at[pl.ds(base, _BPW)], idxv)
        rowsb = (rows0, rows1)
        gsems = (g0, g1)
        wsems = (w0, w1)

        def fire_g(b, d):
            pltpu.async_copy(emb.at[idxv.at[pl.ds(b * _G, _G)]],
                             rowsb[d], gsems[d])

        def wait_g(b, d):
            pltpu.make_async_copy(emb.at[idxv.at[pl.ds(b * _G, _G)]],
                                  rowsb[d], gsems[d]).wait()

        def fire_w(b, d):
            pltpu.async_copy(rowsb[d],
                             out.at[pl.ds(base + b * _G, _G), :], wsems[d])

        def wait_w(b, d):
            pltpu.make_async_copy(rowsb[d],
                                  out.at[pl.ds(base + b * _G, _G), :],
                                  wsems[d]).wait()

        fire_g(0, 0)
        fire_g(1, 1)
        for b in range(_NBATCH):
            d = b % 2
            wait_g(b, d)
            fire_w(b, d)
            if b + 2 < _NBATCH:
                wait_w(b, d)
                fire_g(b + 2, d)
        wait_w(_NBATCH - 2, 0)
        wait_w(_NBATCH - 1, 1)

    return gather


_gather = _make_gather()


_NB = 2000


def _scale0_body(degp, degn, e1, disp, disn, s0p, s0n):
    dgp = degp[...]
    dgn = degn[...]
    dp = jnp.where(dgp > 0, lax.rsqrt(jnp.maximum(dgp, 1.0)), 0.0)
    dn = jnp.where(dgn > 0, lax.rsqrt(jnp.maximum(dgn, 1.0)), 0.0)
    disp[...] = dp
    disn[...] = dn
    e = e1[...]
    s0p[...] = (dp * e).astype(jnp.bfloat16)
    s0n[...] = (dn * e).astype(jnp.bfloat16)


def _scale0(deg_p, deg_n, E1):
    n_blocks = NUM_NODES // _NB
    col = pl.BlockSpec((_NB, 1), lambda i: (i, 0))
    row = pl.BlockSpec((_NB, DIM), lambda i: (i, 0))
    return pl.pallas_call(
        _scale0_body,
        grid=(n_blocks,),
        in_specs=[col, col, row],
        out_specs=[col, col, row, row],
        out_shape=[jax.ShapeDtypeStruct((NUM_NODES, 1), jnp.float32),
                   jax.ShapeDtypeStruct((NUM_NODES, 1), jnp.float32),
                   jax.ShapeDtypeStruct((NUM_NODES, DIM), jnp.bfloat16),
                   jax.ShapeDtypeStruct((NUM_NODES, DIM), jnp.bfloat16)],
    )(deg_p.reshape(NUM_NODES, 1), deg_n.reshape(NUM_NODES, 1), E1)


def _mid_body(t1, d, s1):
    dv = d[...]
    s1[...] = (dv * dv * t1[...].astype(jnp.float32)).astype(jnp.bfloat16)


def _mid(t1, dis):
    n_blocks = NUM_NODES // _NB
    row = pl.BlockSpec((_NB, DIM), lambda i: (i, 0))
    col = pl.BlockSpec((_NB, 1), lambda i: (i, 0))
    return pl.pallas_call(
        _mid_body,
        grid=(n_blocks,),
        in_specs=[row, col],
        out_specs=row,
        out_shape=jax.ShapeDtypeStruct((NUM_NODES, DIM), jnp.bfloat16),
    )(t1, dis)


def _fuse_emb_body(e1, e2, t1p, t2p, dp, t1n, t2n, dn, attn_w, attn_b, q_w,
                   emb_out):
    z_p = (e1[...] + dp[...] * (t1p[...].astype(jnp.float32)
                                + t2p[...].astype(jnp.float32))) * (1.0 / 3.0)
    z_n = (e2[...] + dn[...] * (t1n[...].astype(jnp.float32)
                                + t2n[...].astype(jnp.float32))) * (1.0 / 3.0)
    w_mat = attn_w[...]
    b_vec = attn_b[...]
    q_vec = q_w[...]
    h_p = jnp.tanh(jnp.dot(z_p, w_mat, preferred_element_type=jnp.float32)
                   + b_vec)
    h_n = jnp.tanh(jnp.dot(z_n, w_mat, preferred_element_type=jnp.float32)
                   + b_vec)
    w_p = jnp.dot(h_p, q_vec, preferred_element_type=jnp.float32)
    w_n = jnp.dot(h_n, q_vec, preferred_element_type=jnp.float32)
    mx = jnp.maximum(w_p, w_n)
    ep = jnp.exp(w_p - mx)
    en = jnp.exp(w_n - mx)
    a = ep / (ep + en)
    emb_out[...] = a * z_p + (1.0 - a) * z_n


def _fuse_emb(E1, E2, t1p, t2p, dis_p, t1n, t2n, dis_n, attn_W, attn_b, q_W):
    n_blocks = NUM_NODES // _NB
    row_spec = pl.BlockSpec((_NB, DIM), lambda i: (i, 0))
    col_spec = pl.BlockSpec((_NB, 1), lambda i: (i, 0))
    full2 = pl.BlockSpec((DIM, DIM), lambda i: (0, 0))
    fullb = pl.BlockSpec((1, DIM), lambda i: (0, 0))
    fullq = pl.BlockSpec((DIM, 1), lambda i: (0, 0))
    return pl.pallas_call(
        _fuse_emb_body,
        grid=(n_blocks,),
        in_specs=[row_spec, row_spec, row_spec, row_spec, col_spec,
                  row_spec, row_spec, col_spec, full2, fullb, fullq],
        out_specs=row_spec,
        out_shape=jax.ShapeDtypeStruct((NUM_NODES, DIM), jnp.float32),
    )(E1, E2, t1p, t2p, dis_p, t1n, t2n, dis_n, attn_W,
      attn_b.reshape(1, DIM), q_W)


_BB = 2048


def _log_sigmoid(x):
    t = -jnp.abs(x)
    return jnp.where(x < 0, x, 0.0) - jnp.log1p(jnp.exp(t))


def _loss_body(u_r, v_r, n_r, w_r, out_ref):
    i = pl.program_id(0)

    @pl.when(i == 0)
    def _init():
        out_ref[...] = jnp.zeros((1, 1), jnp.float32)

    u_ = u_r[...]
    v_ = v_r[...]
    nb = n_r[...]
    wv = w_r[...]
    pos = jnp.sum(u_ * v_, axis=1, keepdims=True)
    negsum = jnp.sum(u_[None, :, :] * nb, axis=2)
    coef = -0.5 * jnp.sign(wv) + 1.5
    x = coef * pos - negsum.T
    bpr = jnp.sum(_log_sigmoid(x))
    reg = jnp.sum(u_ * u_) + jnp.sum(v_ * v_) + jnp.sum(nb * nb)
    out_ref[...] += (-bpr + REG * reg).reshape(1, 1)


def _bpr_loss(u_, v_, n3, w):
    n_blocks = B // _BB
    row_spec = pl.BlockSpec((_BB, DIM), lambda i: (i, 0))
    n_spec = pl.BlockSpec((NEG, _BB, DIM), lambda i: (0, i, 0))
    w_spec = pl.BlockSpec((_BB, 1), lambda i: (i, 0))
    out = pl.pallas_call(
        _loss_body,
        grid=(n_blocks,),
        in_specs=[row_spec, row_spec, n_spec, w_spec],
        out_specs=pl.BlockSpec((1, 1), lambda i: (0, 0)),
        out_shape=jax.ShapeDtypeStruct((1, 1), jnp.float32),
    )(u_, v_, n3, w.reshape(B, 1))
    return out[0, 0]


def kernel(E1, E2, attn_W, attn_b, q_W, w, u, v, n, edge_index_p,
           edge_index_n):
    ipad = jnp.full((_C,), -1, jnp.int32)
    zpad = jnp.zeros((_C,), jnp.int32)
    src_p = jnp.concatenate([edge_index_p[0], zpad])
    dst_p = jnp.concatenate([edge_index_p[1], ipad])
    src_n = jnp.concatenate([edge_index_n[0], zpad])
    dst_n = jnp.concatenate([edge_index_n[1], ipad])

    zeros5k = jnp.zeros((5000,), jnp.float32)
    ones1 = jnp.ones((_G,), jnp.float32)
    zrows = jnp.zeros((_G, DIM), jnp.bfloat16)

    deg_p, deg_n = _make_deg()(dst_p, dst_n, zeros5k, ones1)
    dis_p, dis_n, s0p, s0n = _scale0(deg_p, deg_n, E1)

    t1p = _spmm_p(s0p, src_p, dst_p, zrows)
    t2p = _spmm_p(_mid(t1p, dis_p), src_p, dst_p, zrows)
    t1n = _spmm_n(s0n, src_n, dst_n, zrows)
    t2n = _spmm_n(_mid(t1n, dis_n), src_n, dst_n, zrows)

    emb = _fuse_emb(E1, E2, t1p, t2p, dis_p, t1n, t2n, dis_n,
                    attn_W, attn_b, q_W)

    idxall = jnp.concatenate([u, v, n.T.reshape(-1)]).astype(jnp.int32)
    g = _gather(emb, idxall)
    u_ = g[0:B]
    v_ = g[B:2 * B]
    n3 = g[2 * B:].reshape(NEG, B, DIM)
    return _bpr_loss(u_, v_, n3, w)
